# Initial kernel scaffold; baseline (speedup 1.0000x reference)
import jax
import jax.numpy as jnp
from jax.experimental import pallas as pl


def kernel(user_ids, item_ids, gmf_user, gmf_item, mlp_user, mlp_item, W1, b1, W2, b2, Wp, bp):
    raise NotImplementedError("write your pallas kernel here")


# speedup vs baseline: 5.4520x; 5.4520x over previous
import functools

import jax
import jax.numpy as jnp
from jax import lax
from jax.experimental import pallas as pl
from jax.experimental.pallas import tpu as pltpu
from jax.experimental.pallas import tpu_sc as plsc

B = 16384
D = 16
H1 = 32
H2 = 16
NC, NS = 2, 16
NW = NC * NS
BPW = B // NW
CHUNK = 128
NCHUNK = BPW // CHUNK
GROUPS = 8
R = B // GROUPS


def _sc_gather(user_ids, item_ids, gmf_user, gmf_item, mlp_user, mlp_item):
    mesh = plsc.VectorSubcoreMesh(
        core_axis_name="c", subcore_axis_name="s",
        num_cores=NC, num_subcores=NS)
    out = jax.ShapeDtypeStruct((B, D), jnp.float32)

    @functools.partial(
        pl.kernel,
        out_type=(out, out, out, out),
        mesh=mesh,
        scratch_types=[
            pltpu.VMEM((BPW,), jnp.int32),
            pltpu.VMEM((BPW,), jnp.int32),
            pltpu.VMEM((BPW, D), jnp.float32),
            pltpu.VMEM((BPW, D), jnp.float32),
            pltpu.VMEM((BPW, D), jnp.float32),
            pltpu.VMEM((BPW, D), jnp.float32),
            pltpu.SemaphoreType.DMA,
        ],
    )
    def k(uids, iids, t_gu, t_gi, t_mu, t_mi,
          o_gu, o_gi, o_mu, o_mi,
          uidx, iidx, bgu, bgi, bmu, bmi, sem):
        wid = lax.axis_index("s") * NC + lax.axis_index("c")
        base = wid * BPW
        pltpu.sync_copy(uids.at[pl.ds(base, BPW)], uidx)
        pltpu.sync_copy(iids.at[pl.ds(base, BPW)], iidx)
        copies = []
        for t in range(NCHUNK):
            sl = pl.ds(t * CHUNK, CHUNK)
            copies.append(pltpu.async_copy(t_gu.at[uidx.at[sl]], bgu.at[sl], sem))
            copies.append(pltpu.async_copy(t_gi.at[iidx.at[sl]], bgi.at[sl], sem))
            copies.append(pltpu.async_copy(t_mu.at[uidx.at[sl]], bmu.at[sl], sem))
            copies.append(pltpu.async_copy(t_mi.at[iidx.at[sl]], bmi.at[sl], sem))
        for c in copies:
            c.wait()
        pltpu.sync_copy(bgu, o_gu.at[pl.ds(base, BPW)])
        pltpu.sync_copy(bgi, o_gi.at[pl.ds(base, BPW)])
        pltpu.sync_copy(bmu, o_mu.at[pl.ds(base, BPW)])
        pltpu.sync_copy(bmi, o_mi.at[pl.ds(base, BPW)])

    return k(user_ids, item_ids, gmf_user, gmf_item, mlp_user, mlp_item)


def _tc_mlp_body(gu, gi, mu, mi, w1a, w1b, b1, w2, b2, wg, wh, bpr, out_ref):
    f32 = jnp.float32
    h1 = jnp.dot(mu[...], w1a[...], preferred_element_type=f32)
    h1 = h1 + jnp.dot(mi[...], w1b[...], preferred_element_type=f32)
    h1 = jnp.maximum(h1 + b1[...], 0.0)
    h2 = jnp.dot(h1, w2[...], preferred_element_type=f32)
    h2 = jnp.maximum(h2 + b2[...], 0.0)
    g = gu[...] * gi[...]
    o = jnp.dot(g, wg[...], preferred_element_type=f32)
    o = o + jnp.dot(h2, wh[...], preferred_element_type=f32)
    out_ref[...] = o + bpr[...]


def _tc_mlp(gu2, gi2, mu2, mi2, W1A, W1B, b1r, W2bd, b2r, Wg, Wh, bp):
    return pl.pallas_call(
        _tc_mlp_body,
        out_shape=jax.ShapeDtypeStruct((R, GROUPS), jnp.float32),
    )(gu2, gi2, mu2, mi2, W1A, W1B, b1r, W2bd, b2r, Wg, Wh, bp)


def kernel(user_ids, item_ids, gmf_user, gmf_item, mlp_user, mlp_item,
           W1, b1, W2, b2, Wp, bp):
    uids = user_ids.astype(jnp.int32)
    iids = item_ids.astype(jnp.int32)
    gu = jnp.take(gmf_user, uids, axis=0)
    gi = jnp.take(gmf_item, iids, axis=0)
    mu = jnp.take(mlp_user, uids, axis=0)
    mi = jnp.take(mlp_item, iids, axis=0)
    gu2 = gu.reshape(R, GROUPS * D)
    gi2 = gi.reshape(R, GROUPS * D)
    mu2 = mu.reshape(R, GROUPS * D)
    mi2 = mi.reshape(R, GROUPS * D)
    eye = jnp.eye(GROUPS, dtype=jnp.float32)
    W1A = jnp.kron(eye, W1[:D, :])
    W1B = jnp.kron(eye, W1[D:, :])
    W2bd = jnp.kron(eye, W2)
    Wg = jnp.kron(eye, Wp[:D, :])
    Wh = jnp.kron(eye, Wp[D:, :])
    b1r = jnp.tile(b1, GROUPS)
    b2r = jnp.tile(b2, GROUPS)
    o = _tc_mlp(gu2, gi2, mu2, mi2, W1A, W1B, b1r, W2bd, b2r, Wg, Wh, bp)
    return o.reshape(B)
